# baseline (speedup 1.0000x reference)
import functools
import jax
import jax.numpy as jnp
from jax import lax
from jax.experimental import pallas as pl
from jax.experimental.pallas import tpu as pltpu
from jax.experimental.pallas import tpu_sc as plsc

LANES = 16
UNROLL = 4
GROUP = 2
CHUNK = 8192


def _search(x, rep, lane, n_tokens):
    pos_s = lane
    step = n_tokens // 2
    while step >= 2:
        e = plsc.load_gather(rep, [pos_s + (step - 1) * LANES])
        pos_s = jnp.where(e < x, pos_s + step * LANES, pos_s)
        step //= 2
    eprobe = plsc.load_gather(rep, [pos_s])
    m = eprobe < x
    pos_s = jnp.where(m, pos_s + LANES, pos_s)
    wrap = n_tokens * LANES
    other = jnp.where(m, pos_s, (pos_s + (wrap - LANES)) & (wrap - 1))
    eother = plsc.load_gather(rep, [other])
    e0 = jnp.where(m, eother, eprobe)
    em1 = jnp.where(m, eprobe, eother)
    d0 = jnp.abs(e0 - x)
    d1 = jnp.abs(x - em1)
    return (pos_s >> 4) - (d1 < d0).astype(jnp.int32)


def _sc_body(n_tokens, per_w, n_chunks, inp_hbm, tok_hbm, out_hbm,
             edges_v, rep, xb0, xb1, ob0, ob1, si0, si1, so0, so1):
    wid = lax.axis_index("s") * 2 + lax.axis_index("c")
    base = wid * per_w
    pltpu.sync_copy(tok_hbm, edges_v)
    lane = lax.iota(jnp.int32, LANES)

    def build(i, carry):
        e = edges_v[pl.ds(i * LANES, LANES)]
        for j in range(LANES):
            rep[pl.ds((i * LANES + j) * LANES, LANES)] = (
                jnp.broadcast_to(e[j], (LANES,)))
        return carry

    lax.fori_loop(0, n_tokens // LANES, build, 0)

    def in_slice(ci):
        return inp_hbm.at[pl.ds(base + ci * CHUNK, CHUNK)]

    def out_slice(ci):
        return out_hbm.at[pl.ds(base + ci * CHUNK, CHUNK)]

    def compute(xbuf, obuf):
        @plsc.parallel_loop(0, CHUNK, step=LANES * GROUP, unroll=UNROLL)
        def vbody(g0):
            xs = [xbuf[pl.ds(g0 + k * LANES, LANES)] for k in range(GROUP)]
            toks = [_search(xs[k], rep, lane, n_tokens) for k in range(GROUP)]
            for k in range(GROUP):
                obuf[pl.ds(g0 + k * LANES, LANES)] = toks[k]

    pltpu.async_copy(in_slice(0), xb0, si0)
    nsuper = n_chunks // 2

    def super_body(i, carry):
        ci0 = 2 * i
        pltpu.async_copy(in_slice(ci0 + 1), xb1, si1)
        pltpu.make_async_copy(in_slice(ci0), xb0, si0).wait()

        @pl.when(i > 0)
        def _():
            pltpu.make_async_copy(ob0, out_slice(ci0 - 2), so0).wait()

        compute(xb0, ob0)
        pltpu.async_copy(ob0, out_slice(ci0), so0)

        @pl.when(i < nsuper - 1)
        def _():
            pltpu.async_copy(in_slice(ci0 + 2), xb0, si0)

        pltpu.make_async_copy(in_slice(ci0 + 1), xb1, si1).wait()

        @pl.when(i > 0)
        def _():
            pltpu.make_async_copy(ob1, out_slice(ci0 - 1), so1).wait()

        compute(xb1, ob1)
        pltpu.async_copy(ob1, out_slice(ci0 + 1), so1)
        return carry

    lax.fori_loop(0, nsuper, super_body, 0)
    pltpu.make_async_copy(ob0, out_slice(n_chunks - 2), so0).wait()
    pltpu.make_async_copy(ob1, out_slice(n_chunks - 1), so1).wait()


@jax.jit
def kernel(input, token_values):
    n_values = input.shape[0]
    n_tokens = token_values.shape[0]
    n_workers = 32
    per_w = n_values // n_workers
    n_chunks = per_w // CHUNK

    mesh = plsc.VectorSubcoreMesh(core_axis_name="c", subcore_axis_name="s")
    k = functools.partial(
        pl.kernel,
        out_type=jax.ShapeDtypeStruct((n_values,), jnp.int32),
        mesh=mesh,
        scratch_types=[
            pltpu.VMEM((n_tokens,), jnp.float32),
            pltpu.VMEM((n_tokens * LANES,), jnp.float32),
            pltpu.VMEM((CHUNK,), jnp.float32),
            pltpu.VMEM((CHUNK,), jnp.float32),
            pltpu.VMEM((CHUNK,), jnp.int32),
            pltpu.VMEM((CHUNK,), jnp.int32),
            pltpu.SemaphoreType.DMA,
            pltpu.SemaphoreType.DMA,
            pltpu.SemaphoreType.DMA,
            pltpu.SemaphoreType.DMA,
        ],
        compiler_params=pltpu.CompilerParams(needs_layout_passes=False),
    )(functools.partial(_sc_body, n_tokens, per_w, n_chunks))
    return k(input, token_values)


# speedup vs baseline: 1.0342x; 1.0342x over previous
import functools
import jax
import jax.numpy as jnp
from jax import lax
from jax.experimental import pallas as pl
from jax.experimental.pallas import tpu as pltpu
from jax.experimental.pallas import tpu_sc as plsc

LANES = 16
UNROLL = 1
GROUP = 4
CHUNK = 8192


def _search(x, rep, lane, n_tokens):
    pos_s = lane
    step = n_tokens // 2
    while step >= 2:
        e = plsc.load_gather(rep, [pos_s + (step - 1) * LANES])
        pos_s = jnp.where(e < x, pos_s + step * LANES, pos_s)
        step //= 2
    eprobe = plsc.load_gather(rep, [pos_s])
    m = eprobe < x
    pos_s = jnp.where(m, pos_s + LANES, pos_s)
    wrap = n_tokens * LANES
    other = jnp.where(m, pos_s, (pos_s + (wrap - LANES)) & (wrap - 1))
    eother = plsc.load_gather(rep, [other])
    e0 = jnp.where(m, eother, eprobe)
    em1 = jnp.where(m, eprobe, eother)
    d0 = jnp.abs(e0 - x)
    d1 = jnp.abs(x - em1)
    return (pos_s >> 4) - (d1 < d0).astype(jnp.int32)


def _sc_body(n_tokens, per_w, n_chunks, inp_hbm, tok_hbm, out_hbm,
             edges_v, rep, xb0, xb1, ob0, ob1, si0, si1, so0, so1):
    wid = lax.axis_index("s") * 2 + lax.axis_index("c")
    base = wid * per_w
    pltpu.sync_copy(tok_hbm, edges_v)
    lane = lax.iota(jnp.int32, LANES)

    def build(i, carry):
        e = edges_v[pl.ds(i * LANES, LANES)]
        for j in range(LANES):
            rep[pl.ds((i * LANES + j) * LANES, LANES)] = (
                jnp.broadcast_to(e[j], (LANES,)))
        return carry

    lax.fori_loop(0, n_tokens // LANES, build, 0)

    def in_slice(ci):
        return inp_hbm.at[pl.ds(base + ci * CHUNK, CHUNK)]

    def out_slice(ci):
        return out_hbm.at[pl.ds(base + ci * CHUNK, CHUNK)]

    def compute(xbuf, obuf):
        @plsc.parallel_loop(0, CHUNK, step=LANES * GROUP, unroll=UNROLL)
        def vbody(g0):
            xs = [xbuf[pl.ds(g0 + k * LANES, LANES)] for k in range(GROUP)]
            toks = [_search(xs[k], rep, lane, n_tokens) for k in range(GROUP)]
            for k in range(GROUP):
                obuf[pl.ds(g0 + k * LANES, LANES)] = toks[k]

    pltpu.async_copy(in_slice(0), xb0, si0)
    nsuper = n_chunks // 2

    def super_body(i, carry):
        ci0 = 2 * i
        pltpu.async_copy(in_slice(ci0 + 1), xb1, si1)
        pltpu.make_async_copy(in_slice(ci0), xb0, si0).wait()

        @pl.when(i > 0)
        def _():
            pltpu.make_async_copy(ob0, out_slice(ci0 - 2), so0).wait()

        compute(xb0, ob0)
        pltpu.async_copy(ob0, out_slice(ci0), so0)

        @pl.when(i < nsuper - 1)
        def _():
            pltpu.async_copy(in_slice(ci0 + 2), xb0, si0)

        pltpu.make_async_copy(in_slice(ci0 + 1), xb1, si1).wait()

        @pl.when(i > 0)
        def _():
            pltpu.make_async_copy(ob1, out_slice(ci0 - 1), so1).wait()

        compute(xb1, ob1)
        pltpu.async_copy(ob1, out_slice(ci0 + 1), so1)
        return carry

    lax.fori_loop(0, nsuper, super_body, 0)
    pltpu.make_async_copy(ob0, out_slice(n_chunks - 2), so0).wait()
    pltpu.make_async_copy(ob1, out_slice(n_chunks - 1), so1).wait()


@jax.jit
def kernel(input, token_values):
    n_values = input.shape[0]
    n_tokens = token_values.shape[0]
    n_workers = 32
    per_w = n_values // n_workers
    n_chunks = per_w // CHUNK

    mesh = plsc.VectorSubcoreMesh(core_axis_name="c", subcore_axis_name="s")
    k = functools.partial(
        pl.kernel,
        out_type=jax.ShapeDtypeStruct((n_values,), jnp.int32),
        mesh=mesh,
        scratch_types=[
            pltpu.VMEM((n_tokens,), jnp.float32),
            pltpu.VMEM((n_tokens * LANES,), jnp.float32),
            pltpu.VMEM((CHUNK,), jnp.float32),
            pltpu.VMEM((CHUNK,), jnp.float32),
            pltpu.VMEM((CHUNK,), jnp.int32),
            pltpu.VMEM((CHUNK,), jnp.int32),
            pltpu.SemaphoreType.DMA,
            pltpu.SemaphoreType.DMA,
            pltpu.SemaphoreType.DMA,
            pltpu.SemaphoreType.DMA,
        ],
        compiler_params=pltpu.CompilerParams(needs_layout_passes=False),
    )(functools.partial(_sc_body, n_tokens, per_w, n_chunks))
    return k(input, token_values)
